# baseline (speedup 1.0000x reference)
import functools

import jax
import jax.numpy as jnp
import numpy as np
from jax import lax
from jax.experimental import pallas as pl
from jax.experimental.pallas import tpu as pltpu
from jax.experimental.pallas import tpu_sc as plsc

NIC = 1
NF = 24
ITERS = 3
WS = 32
TOPK = 8
RANK = 12
DELTA = 10.0
LEAK = 0.2
KS_LIST = (3, 5, 7)
H = 64
W = 64
N = H * W
M2 = WS * WS
NWIN = 4
_BIG = 3.0e38


def _np_local_mask_inf(ks):
    ii = np.arange(M2)
    yi = ii // WS
    xi = ii % WS
    r = (ks - 1) // 2
    m = (np.abs(yi[:, None] - yi[None, :]) <= r) & (np.abs(xi[:, None] - xi[None, :]) <= r)
    return np.where(m, _BIG, 0.0).astype(np.float32)


_MASK_INF = {ks: _np_local_mask_inf(ks) for ks in KS_LIST}


def _to_windows(h, c):
    return h.reshape(2, WS, 2, WS, c).transpose(0, 2, 1, 3, 4).reshape(NWIN, M2, c)


def _edge_rows(edge):
    e = edge.transpose(2, 0, 1).reshape(TOPK, 2, 2, WS, WS)
    e = e.transpose(0, 1, 3, 2, 4).reshape(TOPK * N)
    return e


def _im2col(h, c, ks):
    r = (ks - 1) // 2
    img = h.reshape(H, W, c)
    xp = jnp.pad(img, ((r, r), (r, r), (0, 0)), mode='reflect')
    cols = [xp[dy:dy + H, dx:dx + W, :] for dy in range(ks) for dx in range(ks)]
    return jnp.concatenate(cols, axis=2).reshape(N, ks * ks * c)


def _sc_gather(table, idx, d):
    b = idx.shape[0]
    nw = 32
    bpw = b // nw
    ch = 128
    mesh = plsc.VectorSubcoreMesh(core_axis_name="c", subcore_axis_name="s")

    @functools.partial(
        pl.kernel, mesh=mesh,
        out_type=jax.ShapeDtypeStruct((b, d), jnp.float32),
        scratch_types=[
            pltpu.VMEM((bpw,), jnp.int32),
            pltpu.VMEM((ch, d), jnp.float32),
            pltpu.VMEM((ch, d), jnp.float32),
            pltpu.SemaphoreType.DMA,
            pltpu.SemaphoreType.DMA,
            pltpu.SemaphoreType.DMA,
        ],
    )
    def gather(table_hbm, idx_hbm, out_hbm, idx_v, rows_a, rows_b, semg,
               sema, semb):
        wid = lax.axis_index("s") * 2 + lax.axis_index("c")
        base = wid * bpw
        pltpu.sync_copy(idx_hbm.at[pl.ds(base, bpw)], idx_v)
        bufs = (rows_a, rows_b)
        osem = (sema, semb)
        ocp = [None, None]
        for j in range(bpw // ch):
            b = j % 2
            if ocp[b] is not None:
                ocp[b].wait()
            pltpu.async_copy(
                table_hbm.at[idx_v.at[pl.ds(j * ch, ch)]],
                bufs[b], semg).wait()
            ocp[b] = pltpu.async_copy(
                bufs[b], out_hbm.at[pl.ds(base + j * ch, ch)], osem[b])
        ocp[0].wait()
        ocp[1].wait()

    return gather(table, idx)


def _center_body(x_ref, o_ref):
    x = x_ref[...]
    o_ref[...] = x - jnp.mean(x)


def _center(x_flat):
    return pl.pallas_call(
        _center_body,
        out_shape=jax.ShapeDtypeStruct((N, 1), jnp.float32),
    )(x_flat)


def _axpby_body(a_ref, b_ref, x_ref, y_ref, o_ref):
    o_ref[...] = a_ref[0, 0] * x_ref[...] + b_ref[0, 0] * y_ref[...]


def _axpby(a, b, x, y):
    a2 = jnp.asarray(a, jnp.float32).reshape(1, 1)
    b2 = jnp.asarray(b, jnp.float32).reshape(1, 1)
    return pl.pallas_call(
        _axpby_body,
        out_shape=jax.ShapeDtypeStruct(x.shape, jnp.float32),
    )(a2, b2, x, y)


def _conv_body(refs, *, has_bias, has_nl, has_bn, act):
    i = 0
    cols = refs[i]; i += 1
    wr = refs[i]; i += 1
    br = gr = bnr = nlr = None
    if has_bias:
        br = refs[i]; i += 1
    if has_nl:
        nlr = refs[i]; i += 1
    if has_bn:
        gr = refs[i]; i += 1
        bnr = refs[i]; i += 1
    o_ref = refs[i]
    y = jnp.dot(cols[...], wr[...], preferred_element_type=jnp.float32)
    if has_nl:
        y = (nlr[...] + y) * 0.5
    if has_bias:
        y = y + br[...]
    if has_bn:
        mu = jnp.mean(y, axis=0, keepdims=True)
        var = jnp.mean((y - mu) ** 2, axis=0, keepdims=True)
        y = (y - mu) * lax.rsqrt(var + 1e-5) * gr[...] + bnr[...]
    if act:
        y = jnp.where(y >= 0, y, LEAK * y)
    o_ref[...] = y


def _conv(cols, w2, bias=None, hnl=None, bn=None, act=False):
    o = w2.shape[1]
    ops = [cols, w2]
    if bias is not None:
        ops.append(bias.reshape(1, o))
    if hnl is not None:
        ops.append(hnl)
    if bn is not None:
        ops.append(bn[0].reshape(1, o))
        ops.append(bn[1].reshape(1, o))
    body = functools.partial(
        _conv_body, has_bias=bias is not None, has_nl=hnl is not None,
        has_bn=bn is not None, act=act)

    def kern(*refs):
        body(refs)

    return pl.pallas_call(
        kern,
        out_shape=jax.ShapeDtypeStruct((N, o), jnp.float32),
    )(*ops)


def _topk_body(hw_ref, minf_ref, o_ref):
    hw = hw_ref[0]
    gram = lax.dot_general(hw, hw, (((1,), (1,)), ((), ())),
                           preferred_element_type=jnp.float32)
    sq = jnp.sum(hw * hw, axis=1, keepdims=True)
    g = sq + jnp.transpose(sq) - 2.0 * gram
    g = g + minf_ref[...]
    iota = lax.broadcasted_iota(jnp.int32, (M2, M2), 1)
    wid = pl.program_id(0)
    gi = (wid // 2) * WS
    gj = (wid % 2) * WS
    for k in range(TOPK):
        mval = jnp.min(g, axis=1, keepdims=True)
        cand = jnp.where(g <= mval, iota, M2 + 1)
        idx = jnp.min(cand, axis=1, keepdims=True)
        o_ref[0, :, k:k + 1] = (gi + idx // WS) * W + gj + idx % WS
        g = jnp.where(iota == idx, _BIG, g)


def _topk(hwin, c, ks):
    minf = jnp.asarray(_MASK_INF[ks])
    return pl.pallas_call(
        _topk_body,
        grid=(NWIN,),
        in_specs=[
            pl.BlockSpec((1, M2, c), lambda w: (w, 0, 0)),
            pl.BlockSpec((M2, M2), lambda w: (0, 0)),
        ],
        out_specs=pl.BlockSpec((1, M2, TOPK), lambda w: (w, 0, 0)),
        out_shape=jax.ShapeDtypeStruct((NWIN, M2, TOPK), jnp.int32),
    )(hwin, minf)


def _ecc_body(vert_ref, hf_ref, w0_ref, b0_ref, wl_ref, bl_ref, wr_ref,
              br_ref, wk_ref, bk_ref, o_ref, *, cin, cout):
    f32 = jnp.float32
    k = pl.program_id(0)
    hf = hf_ref[...]
    vertex = vert_ref[...][:, :cin]
    msum = (lax.broadcasted_iota(jnp.int32, (RANK * cin, RANK), 0) // cin
            == lax.broadcasted_iota(jnp.int32, (RANK * cin, RANK), 1)
            ).astype(f32)
    mexp = (lax.broadcasted_iota(jnp.int32, (RANK, RANK * cout), 0)
            == lax.broadcasted_iota(jnp.int32, (RANK, RANK * cout), 1) // cout
            ).astype(f32)
    msum2 = (lax.broadcasted_iota(jnp.int32, (RANK * cout, cout), 0) % cout
             == lax.broadcasted_iota(jnp.int32, (RANK * cout, cout), 1)
             ).astype(f32)
    label = vertex - hf
    theta = label @ w0_ref[...] + b0_ref[...]
    theta = jnp.where(theta >= 0, theta, LEAK * theta)
    gamma = jnp.exp(-jnp.sum(label * label, axis=1, keepdims=True)
                    * (1.0 / DELTA))
    th_l = theta @ wl_ref[...] + bl_ref[...]
    th_r = theta @ wr_ref[...] + br_ref[...]
    kap = theta @ wk_ref[...] + bk_ref[...]
    vt = jnp.concatenate([vertex] * RANK, axis=1)
    s = jnp.dot(th_r * vt, msum, preferred_element_type=f32)
    t = kap * s
    te = jnp.dot(t, mexp, preferred_element_type=f32)
    outk = jnp.dot(th_l * te, msum2, preferred_element_type=f32)
    contrib = gamma * outk * (1.0 / TOPK)

    @pl.when(k == 0)
    def _():
        o_ref[...] = contrib

    @pl.when(k > 0)
    def _():
        o_ref[...] = o_ref[...] + contrib


def _ecc(vert_rows, hf, p, cin, cout, dpad):
    w0t = p['FC0']['w'].T
    b0 = p['FC0']['b'].reshape(1, cin)
    wl = p['FCL']['w'].reshape(cout, RANK, cin).transpose(1, 0, 2)
    wl = wl.reshape(RANK * cout, cin).T
    bl = p['FCL']['b'].reshape(cout, RANK).T.reshape(1, RANK * cout)
    wr = p['FCR']['w'].reshape(cin, RANK, cin).transpose(1, 0, 2)
    wr = wr.reshape(RANK * cin, cin).T
    br = p['FCR']['b'].reshape(cin, RANK).T.reshape(1, RANK * cin)
    wk = p['FCk']['w'].T
    bk = p['FCk']['b'].reshape(1, RANK)
    full = lambda *s: pl.BlockSpec(s, lambda k: (0,) * len(s))
    return pl.pallas_call(
        functools.partial(_ecc_body, cin=cin, cout=cout),
        grid=(TOPK,),
        in_specs=[
            pl.BlockSpec((N, dpad), lambda k: (k, 0)),
            full(N, cin),
            full(cin, cin), full(1, cin),
            full(cin, RANK * cout), full(1, RANK * cout),
            full(cin, RANK * cin), full(1, RANK * cin),
            full(cin, RANK), full(1, RANK),
        ],
        out_specs=pl.BlockSpec((N, cout), lambda k: (0, 0)),
        out_shape=jax.ShapeDtypeStruct((N, cout), jnp.float32),
    )(vert_rows, hf, w0t, b0, wl, bl, wr, br, wk, bk)


def _pad_lanes(h, c):
    dpad = 128
    return jnp.pad(h, ((0, 0), (0, dpad - c))), dpad


def _graph_conv(p, h, edge_rows, c, cout, ks, bn=None, act=True):
    hpad, dpad = _pad_lanes(h, c)
    vert = _sc_gather(hpad, edge_rows, dpad)
    hnl = _ecc(vert, h, p['ecc'], c, cout, dpad)
    cols = _im2col(h, c, ks)
    w2 = p['conv_w'].transpose(2, 3, 1, 0).reshape(ks * ks * c, cout)
    return _conv(cols, w2, bias=p['bias'].reshape(cout), hnl=hnl,
                 bn=bn, act=act)


def _conv_layer(p, h, c, cout, ks, bn=None, act=False):
    cols = _im2col(h, c, ks)
    w2 = p['w'].transpose(2, 3, 1, 0).reshape(ks * ks * c, cout)
    return _conv(cols, w2, bias=p['b'], bn=bn, act=act)


def _gclayer(p, h, c, block_type, ks):
    pre = 2 if block_type == 'PRE' else 1
    post = 1 if block_type == 'PRE' else 3
    x = h
    for i in range(pre):
        bn = None
        if block_type != 'PRE':
            bn = (p['bnpre'][i]['g'], p['bnpre'][i]['b'])
        x = _conv_layer(p['conv'][i], x, c, c, ks, bn=bn, act=True)
    edge = _topk(_to_windows(x, c), c, ks)
    erows = _edge_rows(edge)
    for i in range(post):
        bn = None
        if block_type == 'LPF':
            bn = (p['bnpost'][i]['g'], p['bnpost'][i]['b'])
        x = _graph_conv(p['gconv'][i], x, erows, c, c, ks, bn=bn, act=True)
    return x


def kernel(x, params):
    xf = x.reshape(N, 1)
    xc = _center(xf)

    nf3 = NF // 3
    feats = []
    for i, ks in enumerate(KS_LIST):
        hi = _conv_layer(params['INCONV'][i], xc, NIC, nf3, ks)
        feats.append(_gclayer(params['PPCONV'][i], hi, nf3, 'PRE', ks))
    z = jnp.concatenate(feats, axis=1)

    hiz = _gclayer(params['HPF'], z, NF, 'HPF', 3)
    alpha = params['alpha']
    beta = params['beta']
    for i in range(ITERS):
        z = _axpby(1.0 - alpha[i], beta[i], z, hiz)
        z = _axpby(1.0, 1.0, z, _gclayer(params['LPF'][i], z, NF, 'LPF', 3))
    z = _axpby(1.0 - alpha[-1], beta[-1], z, hiz)

    edge = _topk(_to_windows(z, NF), NF, 3)
    zo = _graph_conv(params['GCout'], z, _edge_rows(edge), NF, NIC, 3,
                     act=False)

    out = _axpby(1.0, 1.0, xf, zo)
    return out.reshape(1, NIC, H, W)


# speedup vs baseline: 1.0128x; 1.0128x over previous
import functools

import jax
import jax.numpy as jnp
import numpy as np
from jax import lax
from jax.experimental import pallas as pl
from jax.experimental.pallas import tpu as pltpu
from jax.experimental.pallas import tpu_sc as plsc

NIC = 1
NF = 24
ITERS = 3
WS = 32
TOPK = 8
RANK = 12
DELTA = 10.0
LEAK = 0.2
KS_LIST = (3, 5, 7)
H = 64
W = 64
N = H * W
M2 = WS * WS
NWIN = 4
_BIG = 3.0e38


def _np_local_mask_inf(ks):
    ii = np.arange(M2)
    yi = ii // WS
    xi = ii % WS
    r = (ks - 1) // 2
    m = (np.abs(yi[:, None] - yi[None, :]) <= r) & (np.abs(xi[:, None] - xi[None, :]) <= r)
    return np.where(m, _BIG, 0.0).astype(np.float32)


_MASK_INF = {ks: _np_local_mask_inf(ks) for ks in KS_LIST}


def _to_windows(h, c):
    return h.reshape(2, WS, 2, WS, c).transpose(0, 2, 1, 3, 4).reshape(NWIN, M2, c)


def _edge_rows(edge):
    e = edge.transpose(2, 0, 1).reshape(TOPK, 2, 2, WS, WS)
    e = e.transpose(0, 1, 3, 2, 4).reshape(TOPK * N)
    return e


def _im2col(h, c, ks):
    r = (ks - 1) // 2
    img = h.reshape(H, W, c)
    xp = jnp.pad(img, ((r, r), (r, r), (0, 0)), mode='reflect')
    cols = [xp[dy:dy + H, dx:dx + W, :] for dy in range(ks) for dx in range(ks)]
    return jnp.concatenate(cols, axis=2).reshape(N, ks * ks * c)


def _sc_gather(table, idx, d):
    b = idx.shape[0]
    nw = 32
    bpw = b // nw
    ch = 128
    mesh = plsc.VectorSubcoreMesh(core_axis_name="c", subcore_axis_name="s")

    @functools.partial(
        pl.kernel, mesh=mesh,
        out_type=jax.ShapeDtypeStruct((b, d), jnp.float32),
        scratch_types=[
            pltpu.VMEM((bpw,), jnp.int32),
            pltpu.VMEM((ch, d), jnp.float32),
            pltpu.VMEM((ch, d), jnp.float32),
            pltpu.SemaphoreType.DMA,
            pltpu.SemaphoreType.DMA,
            pltpu.SemaphoreType.DMA,
        ],
    )
    def gather(table_hbm, idx_hbm, out_hbm, idx_v, rows_a, rows_b, semg,
               sema, semb):
        wid = lax.axis_index("s") * 2 + lax.axis_index("c")
        base = wid * bpw
        pltpu.sync_copy(idx_hbm.at[pl.ds(base, bpw)], idx_v)
        bufs = (rows_a, rows_b)
        osem = (sema, semb)
        ocp = [None, None]
        for j in range(bpw // ch):
            b = j % 2
            if ocp[b] is not None:
                ocp[b].wait()
            pltpu.async_copy(
                table_hbm.at[idx_v.at[pl.ds(j * ch, ch)]],
                bufs[b], semg).wait()
            ocp[b] = pltpu.async_copy(
                bufs[b], out_hbm.at[pl.ds(base + j * ch, ch)], osem[b])
        ocp[0].wait()
        ocp[1].wait()

    return gather(table, idx)


def _center_body(x_ref, o_ref):
    x = x_ref[...]
    o_ref[...] = x - jnp.mean(x)


def _center(x_flat):
    return pl.pallas_call(
        _center_body,
        out_shape=jax.ShapeDtypeStruct((N, 1), jnp.float32),
    )(x_flat)


def _axpby_body(a_ref, b_ref, x_ref, y_ref, o_ref):
    o_ref[...] = a_ref[0, 0] * x_ref[...] + b_ref[0, 0] * y_ref[...]


def _axpby(a, b, x, y):
    a2 = jnp.asarray(a, jnp.float32).reshape(1, 1)
    b2 = jnp.asarray(b, jnp.float32).reshape(1, 1)
    return pl.pallas_call(
        _axpby_body,
        out_shape=jax.ShapeDtypeStruct(x.shape, jnp.float32),
    )(a2, b2, x, y)


def _conv_body(refs, *, has_bias, has_nl, has_bn, act):
    i = 0
    cols = refs[i]; i += 1
    wr = refs[i]; i += 1
    br = gr = bnr = nlr = None
    if has_bias:
        br = refs[i]; i += 1
    if has_nl:
        nlr = refs[i]; i += 1
    if has_bn:
        gr = refs[i]; i += 1
        bnr = refs[i]; i += 1
    o_ref = refs[i]
    y = jnp.dot(cols[...], wr[...], preferred_element_type=jnp.float32)
    if has_nl:
        y = (nlr[...] + y) * 0.5
    if has_bias:
        y = y + br[...]
    if has_bn:
        mu = jnp.mean(y, axis=0, keepdims=True)
        var = jnp.mean((y - mu) ** 2, axis=0, keepdims=True)
        y = (y - mu) * lax.rsqrt(var + 1e-5) * gr[...] + bnr[...]
    if act:
        y = jnp.where(y >= 0, y, LEAK * y)
    o_ref[...] = y


def _conv(cols, w2, bias=None, hnl=None, bn=None, act=False):
    o = w2.shape[1]
    ops = [cols, w2]
    if bias is not None:
        ops.append(bias.reshape(1, o))
    if hnl is not None:
        ops.append(hnl)
    if bn is not None:
        ops.append(bn[0].reshape(1, o))
        ops.append(bn[1].reshape(1, o))
    body = functools.partial(
        _conv_body, has_bias=bias is not None, has_nl=hnl is not None,
        has_bn=bn is not None, act=act)

    def kern(*refs):
        body(refs)

    return pl.pallas_call(
        kern,
        out_shape=jax.ShapeDtypeStruct((N, o), jnp.float32),
    )(*ops)


def _topk_body(hw_ref, minf_ref, o_ref):
    hw = hw_ref[0]
    gram = lax.dot_general(hw, hw, (((1,), (1,)), ((), ())),
                           preferred_element_type=jnp.float32)
    sq = jnp.sum(hw * hw, axis=1, keepdims=True)
    g = sq + jnp.transpose(sq) - 2.0 * gram
    g = g + minf_ref[...]
    iota = lax.broadcasted_iota(jnp.int32, (M2, M2), 1)
    wid = pl.program_id(0)
    gi = (wid // 2) * WS
    gj = (wid % 2) * WS
    for k in range(TOPK):
        mval = jnp.min(g, axis=1, keepdims=True)
        cand = jnp.where(g <= mval, iota, M2 + 1)
        idx = jnp.min(cand, axis=1, keepdims=True)
        o_ref[0, :, k:k + 1] = (gi + idx // WS) * W + gj + idx % WS
        g = jnp.where(iota == idx, _BIG, g)


def _topk(hwin, c, ks):
    minf = jnp.asarray(_MASK_INF[ks])
    return pl.pallas_call(
        _topk_body,
        grid=(NWIN,),
        in_specs=[
            pl.BlockSpec((1, M2, c), lambda w: (w, 0, 0)),
            pl.BlockSpec((M2, M2), lambda w: (0, 0)),
        ],
        out_specs=pl.BlockSpec((1, M2, TOPK), lambda w: (w, 0, 0)),
        out_shape=jax.ShapeDtypeStruct((NWIN, M2, TOPK), jnp.int32),
    )(hwin, minf)


def _ecc_body(vert_ref, hf_ref, w0_ref, b0_ref, wl_ref, bl_ref, wr_ref,
              br_ref, wk_ref, bk_ref, cols_ref, w2_ref, gb_ref, g_ref,
              bn_ref, o_ref, *, cin, cout, has_bn, act):
    f32 = jnp.float32
    k = pl.program_id(0)
    hf = hf_ref[...]
    vertex = vert_ref[...][:, :cin]
    msum = (lax.broadcasted_iota(jnp.int32, (RANK * cin, RANK), 0) // cin
            == lax.broadcasted_iota(jnp.int32, (RANK * cin, RANK), 1)
            ).astype(f32)
    mexp = (lax.broadcasted_iota(jnp.int32, (RANK, RANK * cout), 0)
            == lax.broadcasted_iota(jnp.int32, (RANK, RANK * cout), 1) // cout
            ).astype(f32)
    msum2 = (lax.broadcasted_iota(jnp.int32, (RANK * cout, cout), 0) % cout
             == lax.broadcasted_iota(jnp.int32, (RANK * cout, cout), 1)
             ).astype(f32)
    label = vertex - hf
    theta = label @ w0_ref[...] + b0_ref[...]
    theta = jnp.where(theta >= 0, theta, LEAK * theta)
    gamma = jnp.exp(-jnp.sum(label * label, axis=1, keepdims=True)
                    * (1.0 / DELTA))
    th_l = theta @ wl_ref[...] + bl_ref[...]
    th_r = theta @ wr_ref[...] + br_ref[...]
    kap = theta @ wk_ref[...] + bk_ref[...]
    vt = jnp.concatenate([vertex] * RANK, axis=1)
    s = jnp.dot(th_r * vt, msum, preferred_element_type=f32)
    t = kap * s
    te = jnp.dot(t, mexp, preferred_element_type=f32)
    outk = jnp.dot(th_l * te, msum2, preferred_element_type=f32)
    contrib = gamma * outk * (1.0 / TOPK)

    @pl.when(k == 0)
    def _():
        o_ref[...] = contrib

    @pl.when((k > 0) & (k < TOPK - 1))
    def _():
        o_ref[...] = o_ref[...] + contrib

    @pl.when(k == TOPK - 1)
    def _():
        hnl = o_ref[...] + contrib
        hl = jnp.dot(cols_ref[...], w2_ref[...], preferred_element_type=f32)
        y = (hnl + hl) * 0.5 + gb_ref[...]
        if has_bn:
            mu = jnp.mean(y, axis=0, keepdims=True)
            var = jnp.mean((y - mu) ** 2, axis=0, keepdims=True)
            y = (y - mu) * lax.rsqrt(var + 1e-5) * g_ref[...] + bn_ref[...]
        if act:
            y = jnp.where(y >= 0, y, LEAK * y)
        o_ref[...] = y


def _ecc_conv(vert_rows, hf, p, cin, cout, dpad, cols, w2, bn, act):
    pe = p['ecc']
    w0t = pe['FC0']['w'].T
    b0 = pe['FC0']['b'].reshape(1, cin)
    wl = pe['FCL']['w'].reshape(cout, RANK, cin).transpose(1, 0, 2)
    wl = wl.reshape(RANK * cout, cin).T
    bl = pe['FCL']['b'].reshape(cout, RANK).T.reshape(1, RANK * cout)
    wr = pe['FCR']['w'].reshape(cin, RANK, cin).transpose(1, 0, 2)
    wr = wr.reshape(RANK * cin, cin).T
    br = pe['FCR']['b'].reshape(cin, RANK).T.reshape(1, RANK * cin)
    wk = pe['FCk']['w'].T
    bk = pe['FCk']['b'].reshape(1, RANK)
    kk = cols.shape[1]
    gb = p['bias'].reshape(1, cout)
    if bn is None:
        g = jnp.ones((1, cout), jnp.float32)
        bnb = jnp.zeros((1, cout), jnp.float32)
    else:
        g = bn[0].reshape(1, cout)
        bnb = bn[1].reshape(1, cout)
    full = lambda *s: pl.BlockSpec(s, lambda k: (0,) * len(s))
    return pl.pallas_call(
        functools.partial(_ecc_body, cin=cin, cout=cout,
                          has_bn=bn is not None, act=act),
        grid=(TOPK,),
        in_specs=[
            pl.BlockSpec((N, dpad), lambda k: (k, 0)),
            full(N, cin),
            full(cin, cin), full(1, cin),
            full(cin, RANK * cout), full(1, RANK * cout),
            full(cin, RANK * cin), full(1, RANK * cin),
            full(cin, RANK), full(1, RANK),
            full(N, kk), full(kk, cout), full(1, cout),
            full(1, cout), full(1, cout),
        ],
        out_specs=pl.BlockSpec((N, cout), lambda k: (0, 0)),
        out_shape=jax.ShapeDtypeStruct((N, cout), jnp.float32),
    )(vert_rows, hf, w0t, b0, wl, bl, wr, br, wk, bk,
      cols, w2, gb, g, bnb)


def _pad_lanes(h, c):
    dpad = 128
    return jnp.pad(h, ((0, 0), (0, dpad - c))), dpad


def _graph_conv(p, h, edge_rows, c, cout, ks, bn=None, act=True):
    hpad, dpad = _pad_lanes(h, c)
    vert = _sc_gather(hpad, edge_rows, dpad)
    cols = _im2col(h, c, ks)
    w2 = p['conv_w'].transpose(2, 3, 1, 0).reshape(ks * ks * c, cout)
    return _ecc_conv(vert, h, p, c, cout, dpad, cols, w2, bn, act)


def _conv_layer(p, h, c, cout, ks, bn=None, act=False):
    cols = _im2col(h, c, ks)
    w2 = p['w'].transpose(2, 3, 1, 0).reshape(ks * ks * c, cout)
    return _conv(cols, w2, bias=p['b'], bn=bn, act=act)


def _gclayer(p, h, c, block_type, ks):
    pre = 2 if block_type == 'PRE' else 1
    post = 1 if block_type == 'PRE' else 3
    x = h
    for i in range(pre):
        bn = None
        if block_type != 'PRE':
            bn = (p['bnpre'][i]['g'], p['bnpre'][i]['b'])
        x = _conv_layer(p['conv'][i], x, c, c, ks, bn=bn, act=True)
    edge = _topk(_to_windows(x, c), c, ks)
    erows = _edge_rows(edge)
    for i in range(post):
        bn = None
        if block_type == 'LPF':
            bn = (p['bnpost'][i]['g'], p['bnpost'][i]['b'])
        x = _graph_conv(p['gconv'][i], x, erows, c, c, ks, bn=bn, act=True)
    return x


def kernel(x, params):
    xf = x.reshape(N, 1)
    xc = _center(xf)

    nf3 = NF // 3
    feats = []
    for i, ks in enumerate(KS_LIST):
        hi = _conv_layer(params['INCONV'][i], xc, NIC, nf3, ks)
        feats.append(_gclayer(params['PPCONV'][i], hi, nf3, 'PRE', ks))
    z = jnp.concatenate(feats, axis=1)

    hiz = _gclayer(params['HPF'], z, NF, 'HPF', 3)
    alpha = params['alpha']
    beta = params['beta']
    for i in range(ITERS):
        z = _axpby(1.0 - alpha[i], beta[i], z, hiz)
        z = _axpby(1.0, 1.0, z, _gclayer(params['LPF'][i], z, NF, 'LPF', 3))
    z = _axpby(1.0 - alpha[-1], beta[-1], z, hiz)

    edge = _topk(_to_windows(z, NF), NF, 3)
    zo = _graph_conv(params['GCout'], z, _edge_rows(edge), NF, NIC, 3,
                     act=False)

    out = _axpby(1.0, 1.0, xf, zo)
    return out.reshape(1, NIC, H, W)


# speedup vs baseline: 1.0276x; 1.0146x over previous
import functools

import jax
import jax.numpy as jnp
import numpy as np
from jax import lax
from jax.experimental import pallas as pl
from jax.experimental.pallas import tpu as pltpu
from jax.experimental.pallas import tpu_sc as plsc

NIC = 1
NF = 24
ITERS = 3
WS = 32
TOPK = 8
RANK = 12
DELTA = 10.0
LEAK = 0.2
KS_LIST = (3, 5, 7)
H = 64
W = 64
N = H * W
M2 = WS * WS
NWIN = 4
_BIG = 3.0e38


def _np_local_mask_inf(ks):
    ii = np.arange(M2)
    yi = ii // WS
    xi = ii % WS
    r = (ks - 1) // 2
    m = (np.abs(yi[:, None] - yi[None, :]) <= r) & (np.abs(xi[:, None] - xi[None, :]) <= r)
    return np.where(m, _BIG, 0.0).astype(np.float32)


_MASK_INF = {ks: _np_local_mask_inf(ks) for ks in KS_LIST}


def _to_windows(h, c):
    return h.reshape(2, WS, 2, WS, c).transpose(0, 2, 1, 3, 4).reshape(NWIN, M2, c)


def _edge_rows(edge):
    e = edge.transpose(2, 0, 1).reshape(TOPK, 2, 2, WS, WS)
    e = e.transpose(0, 1, 3, 2, 4).reshape(TOPK * N)
    return e


def _im2col(h, c, ks):
    r = (ks - 1) // 2
    img = h.reshape(H, W, c)
    xp = jnp.pad(img, ((r, r), (r, r), (0, 0)), mode='reflect')
    cols = [xp[dy:dy + H, dx:dx + W, :] for dy in range(ks) for dx in range(ks)]
    return jnp.concatenate(cols, axis=2).reshape(N, ks * ks * c)


def _sc_gather(table, idx, d):
    b = idx.shape[0]
    nw = 32
    bpw = b // nw
    ch = 128
    mesh = plsc.VectorSubcoreMesh(core_axis_name="c", subcore_axis_name="s")

    @functools.partial(
        pl.kernel, mesh=mesh,
        out_type=jax.ShapeDtypeStruct((b, d), jnp.float32),
        scratch_types=[
            pltpu.VMEM((bpw,), jnp.int32),
            pltpu.VMEM((ch, d), jnp.float32),
            pltpu.VMEM((ch, d), jnp.float32),
            pltpu.SemaphoreType.DMA,
            pltpu.SemaphoreType.DMA,
            pltpu.SemaphoreType.DMA,
        ],
    )
    def gather(table_hbm, idx_hbm, out_hbm, idx_v, rows_a, rows_b, semg,
               sema, semb):
        wid = lax.axis_index("s") * 2 + lax.axis_index("c")
        base = wid * bpw
        pltpu.sync_copy(idx_hbm.at[pl.ds(base, bpw)], idx_v)
        bufs = (rows_a, rows_b)
        osem = (sema, semb)
        ocp = [None, None]
        for j in range(bpw // ch):
            b = j % 2
            if ocp[b] is not None:
                ocp[b].wait()
            pltpu.async_copy(
                table_hbm.at[idx_v.at[pl.ds(j * ch, ch)]],
                bufs[b], semg).wait()
            ocp[b] = pltpu.async_copy(
                bufs[b], out_hbm.at[pl.ds(base + j * ch, ch)], osem[b])
        ocp[0].wait()
        ocp[1].wait()

    return gather(table, idx)


def _center_body(x_ref, o_ref):
    x = x_ref[...]
    o_ref[...] = x - jnp.mean(x)


def _center(x_flat):
    return pl.pallas_call(
        _center_body,
        out_shape=jax.ShapeDtypeStruct((N, 1), jnp.float32),
    )(x_flat)


def _axpby_body(a_ref, b_ref, x_ref, y_ref, o_ref):
    o_ref[...] = a_ref[0, 0] * x_ref[...] + b_ref[0, 0] * y_ref[...]


def _axpby(a, b, x, y):
    a2 = jnp.asarray(a, jnp.float32).reshape(1, 1)
    b2 = jnp.asarray(b, jnp.float32).reshape(1, 1)
    return pl.pallas_call(
        _axpby_body,
        out_shape=jax.ShapeDtypeStruct(x.shape, jnp.float32),
    )(a2, b2, x, y)


def _conv_body(refs, *, has_bias, has_nl, has_bn, act):
    i = 0
    cols = refs[i]; i += 1
    wr = refs[i]; i += 1
    br = gr = bnr = nlr = None
    if has_bias:
        br = refs[i]; i += 1
    if has_nl:
        nlr = refs[i]; i += 1
    if has_bn:
        gr = refs[i]; i += 1
        bnr = refs[i]; i += 1
    o_ref = refs[i]
    y = jnp.dot(cols[...], wr[...], preferred_element_type=jnp.float32)
    if has_nl:
        y = (nlr[...] + y) * 0.5
    if has_bias:
        y = y + br[...]
    if has_bn:
        mu = jnp.mean(y, axis=0, keepdims=True)
        var = jnp.mean((y - mu) ** 2, axis=0, keepdims=True)
        y = (y - mu) * lax.rsqrt(var + 1e-5) * gr[...] + bnr[...]
    if act:
        y = jnp.where(y >= 0, y, LEAK * y)
    o_ref[...] = y


def _conv(cols, w2, bias=None, hnl=None, bn=None, act=False):
    o = w2.shape[1]
    ops = [cols, w2]
    if bias is not None:
        ops.append(bias.reshape(1, o))
    if hnl is not None:
        ops.append(hnl)
    if bn is not None:
        ops.append(bn[0].reshape(1, o))
        ops.append(bn[1].reshape(1, o))
    body = functools.partial(
        _conv_body, has_bias=bias is not None, has_nl=hnl is not None,
        has_bn=bn is not None, act=act)

    def kern(*refs):
        body(refs)

    return pl.pallas_call(
        kern,
        out_shape=jax.ShapeDtypeStruct((N, o), jnp.float32),
    )(*ops)


def _topk_body(hw_ref, minf_ref, o_ref, v_ref):
    hw = hw_ref[0]
    gram = lax.dot_general(hw, hw, (((1,), (1,)), ((), ())),
                           preferred_element_type=jnp.float32)
    sq = jnp.sum(hw * hw, axis=1, keepdims=True)
    g = sq + jnp.transpose(sq) - 2.0 * gram
    g = g + minf_ref[...]
    iota = lax.broadcasted_iota(jnp.int32, (M2, M2), 1)
    wid = pl.program_id(0)
    gi = (wid // 2) * WS
    gj = (wid % 2) * WS
    for k in range(TOPK):
        mval = jnp.min(g, axis=1, keepdims=True)
        cand = jnp.where(g <= mval, iota, M2 + 1)
        idx = jnp.min(cand, axis=1, keepdims=True)
        o_ref[0, :, k:k + 1] = (gi + idx // WS) * W + gj + idx % WS
        oh = iota == idx
        v_ref[k, 0] = jnp.dot(oh.astype(jnp.float32), hw,
                              preferred_element_type=jnp.float32)
        g = jnp.where(oh, _BIG, g)


def _topk(hwin, c, ks):
    minf = jnp.asarray(_MASK_INF[ks])
    return pl.pallas_call(
        _topk_body,
        grid=(NWIN,),
        in_specs=[
            pl.BlockSpec((1, M2, c), lambda w: (w, 0, 0)),
            pl.BlockSpec((M2, M2), lambda w: (0, 0)),
        ],
        out_specs=[
            pl.BlockSpec((1, M2, TOPK), lambda w: (w, 0, 0)),
            pl.BlockSpec((TOPK, 1, M2, c), lambda w: (0, w, 0, 0)),
        ],
        out_shape=[
            jax.ShapeDtypeStruct((NWIN, M2, TOPK), jnp.int32),
            jax.ShapeDtypeStruct((TOPK, NWIN, M2, c), jnp.float32),
        ],
    )(hwin, minf)


def _ecc_body(vert_ref, hf_ref, w0_ref, b0_ref, wl_ref, bl_ref, wr_ref,
              br_ref, wk_ref, bk_ref, cols_ref, w2_ref, gb_ref, g_ref,
              bn_ref, o_ref, *, cin, cout, has_bn, act):
    f32 = jnp.float32
    k = pl.program_id(0)
    hf = hf_ref[...]
    vertex = vert_ref[...][:, :cin]
    msum = (lax.broadcasted_iota(jnp.int32, (RANK * cin, RANK), 0) // cin
            == lax.broadcasted_iota(jnp.int32, (RANK * cin, RANK), 1)
            ).astype(f32)
    mexp = (lax.broadcasted_iota(jnp.int32, (RANK, RANK * cout), 0)
            == lax.broadcasted_iota(jnp.int32, (RANK, RANK * cout), 1) // cout
            ).astype(f32)
    msum2 = (lax.broadcasted_iota(jnp.int32, (RANK * cout, cout), 0) % cout
             == lax.broadcasted_iota(jnp.int32, (RANK * cout, cout), 1)
             ).astype(f32)
    label = vertex - hf
    theta = label @ w0_ref[...] + b0_ref[...]
    theta = jnp.where(theta >= 0, theta, LEAK * theta)
    gamma = jnp.exp(-jnp.sum(label * label, axis=1, keepdims=True)
                    * (1.0 / DELTA))
    th_l = theta @ wl_ref[...] + bl_ref[...]
    th_r = theta @ wr_ref[...] + br_ref[...]
    kap = theta @ wk_ref[...] + bk_ref[...]
    vt = jnp.concatenate([vertex] * RANK, axis=1)
    s = jnp.dot(th_r * vt, msum, preferred_element_type=f32)
    t = kap * s
    te = jnp.dot(t, mexp, preferred_element_type=f32)
    outk = jnp.dot(th_l * te, msum2, preferred_element_type=f32)
    contrib = gamma * outk * (1.0 / TOPK)

    @pl.when(k == 0)
    def _():
        o_ref[...] = contrib

    @pl.when((k > 0) & (k < TOPK - 1))
    def _():
        o_ref[...] = o_ref[...] + contrib

    @pl.when(k == TOPK - 1)
    def _():
        hnl = o_ref[...] + contrib
        hl = jnp.dot(cols_ref[...], w2_ref[...], preferred_element_type=f32)
        y = (hnl + hl) * 0.5 + gb_ref[...]
        if has_bn:
            mu = jnp.mean(y, axis=0, keepdims=True)
            var = jnp.mean((y - mu) ** 2, axis=0, keepdims=True)
            y = (y - mu) * lax.rsqrt(var + 1e-5) * g_ref[...] + bn_ref[...]
        if act:
            y = jnp.where(y >= 0, y, LEAK * y)
        o_ref[...] = y


def _ecc_conv(vert_rows, hf, p, cin, cout, dpad, cols, w2, bn, act):
    pe = p['ecc']
    w0t = pe['FC0']['w'].T
    b0 = pe['FC0']['b'].reshape(1, cin)
    wl = pe['FCL']['w'].reshape(cout, RANK, cin).transpose(1, 0, 2)
    wl = wl.reshape(RANK * cout, cin).T
    bl = pe['FCL']['b'].reshape(cout, RANK).T.reshape(1, RANK * cout)
    wr = pe['FCR']['w'].reshape(cin, RANK, cin).transpose(1, 0, 2)
    wr = wr.reshape(RANK * cin, cin).T
    br = pe['FCR']['b'].reshape(cin, RANK).T.reshape(1, RANK * cin)
    wk = pe['FCk']['w'].T
    bk = pe['FCk']['b'].reshape(1, RANK)
    kk = cols.shape[1]
    gb = p['bias'].reshape(1, cout)
    if bn is None:
        g = jnp.ones((1, cout), jnp.float32)
        bnb = jnp.zeros((1, cout), jnp.float32)
    else:
        g = bn[0].reshape(1, cout)
        bnb = bn[1].reshape(1, cout)
    full = lambda *s: pl.BlockSpec(s, lambda k: (0,) * len(s))
    return pl.pallas_call(
        functools.partial(_ecc_body, cin=cin, cout=cout,
                          has_bn=bn is not None, act=act),
        grid=(TOPK,),
        in_specs=[
            pl.BlockSpec((N, dpad), lambda k: (k, 0)),
            full(N, cin),
            full(cin, cin), full(1, cin),
            full(cin, RANK * cout), full(1, RANK * cout),
            full(cin, RANK * cin), full(1, RANK * cin),
            full(cin, RANK), full(1, RANK),
            full(N, kk), full(kk, cout), full(1, cout),
            full(1, cout), full(1, cout),
        ],
        out_specs=pl.BlockSpec((N, cout), lambda k: (0, 0)),
        out_shape=jax.ShapeDtypeStruct((N, cout), jnp.float32),
    )(vert_rows, hf, w0t, b0, wl, bl, wr, br, wk, bk,
      cols, w2, gb, g, bnb)


def _pad_lanes(h, c):
    dpad = 128
    return jnp.pad(h, ((0, 0), (0, dpad - c))), dpad


def _graph_conv(p, h, edge_rows, c, cout, ks, bn=None, act=True):
    hpad, dpad = _pad_lanes(h, c)
    vert = _sc_gather(hpad, edge_rows, dpad)
    cols = _im2col(h, c, ks)
    w2 = p['conv_w'].transpose(2, 3, 1, 0).reshape(ks * ks * c, cout)
    return _ecc_conv(vert, h, p, c, cout, dpad, cols, w2, bn, act)


def _graph_conv_w(p, h, vertw, c, cout, ks, bn=None, act=True):
    hf_w = _to_windows(h, c).reshape(N, c)
    cols_w = _to_windows(_im2col(h, c, ks), ks * ks * c).reshape(N, -1)
    w2 = p['conv_w'].transpose(2, 3, 1, 0).reshape(ks * ks * c, cout)
    vert_rows = vertw.reshape(TOPK * N, c)
    y_w = _ecc_conv(vert_rows, hf_w, p, c, cout, c, cols_w, w2, bn, act)
    return y_w.reshape(2, 2, WS, WS, cout).transpose(0, 2, 1, 3, 4).reshape(N, cout)


def _conv_layer(p, h, c, cout, ks, bn=None, act=False):
    cols = _im2col(h, c, ks)
    w2 = p['w'].transpose(2, 3, 1, 0).reshape(ks * ks * c, cout)
    return _conv(cols, w2, bias=p['b'], bn=bn, act=act)


def _gclayer(p, h, c, block_type, ks):
    pre = 2 if block_type == 'PRE' else 1
    post = 1 if block_type == 'PRE' else 3
    x = h
    for i in range(pre):
        bn = None
        if block_type != 'PRE':
            bn = (p['bnpre'][i]['g'], p['bnpre'][i]['b'])
        x = _conv_layer(p['conv'][i], x, c, c, ks, bn=bn, act=True)
    edge, vertw = _topk(_to_windows(x, c), c, ks)
    erows = _edge_rows(edge) if post > 1 else None
    for i in range(post):
        bn = None
        if block_type == 'LPF':
            bn = (p['bnpost'][i]['g'], p['bnpost'][i]['b'])
        if i == 0:
            x = _graph_conv_w(p['gconv'][i], x, vertw, c, c, ks, bn=bn,
                              act=True)
        else:
            x = _graph_conv(p['gconv'][i], x, erows, c, c, ks, bn=bn,
                            act=True)
    return x


def kernel(x, params):
    xf = x.reshape(N, 1)
    xc = _center(xf)

    nf3 = NF // 3
    feats = []
    for i, ks in enumerate(KS_LIST):
        hi = _conv_layer(params['INCONV'][i], xc, NIC, nf3, ks)
        feats.append(_gclayer(params['PPCONV'][i], hi, nf3, 'PRE', ks))
    z = jnp.concatenate(feats, axis=1)

    hiz = _gclayer(params['HPF'], z, NF, 'HPF', 3)
    alpha = params['alpha']
    beta = params['beta']
    for i in range(ITERS):
        z = _axpby(1.0 - alpha[i], beta[i], z, hiz)
        z = _axpby(1.0, 1.0, z, _gclayer(params['LPF'][i], z, NF, 'LPF', 3))
    z = _axpby(1.0 - alpha[-1], beta[-1], z, hiz)

    edge, vertw = _topk(_to_windows(z, NF), NF, 3)
    zo = _graph_conv_w(params['GCout'], z, vertw, NF, NIC, 3, act=False)

    out = _axpby(1.0, 1.0, xf, zo)
    return out.reshape(1, NIC, H, W)
